# Initial kernel scaffold; baseline (speedup 1.0000x reference)
import jax
import jax.numpy as jnp
from jax.experimental import pallas as pl


def kernel(hidden_states, W1, b1, W2, b2, ln_g, ln_b, Wq, bq, Wg, bg, Wf1, bf1, Wf2, bf2):
    raise NotImplementedError("write your pallas kernel here")


# speedup vs baseline: 3.4286x; 3.4286x over previous
import functools

import jax
import jax.numpy as jnp
from jax.experimental import pallas as pl
from jax.experimental.pallas import tpu as pltpu

H = 2048
EMB = 128
CHUNK = 64
OVERLAP = 16
TOPK = 3
STRIDE = CHUNK - OVERLAP

_POOL_ROWS = 64

_INV_SQRT2 = 0.7071067811865476


def _gelu_exact(x):
    return 0.5 * x * (1.0 + jax.lax.erf(x * _INV_SQRT2))


def _pool_body(hs_ref, x_ref, *, n_chunks, seq_len):
    hsb = hs_ref[0]
    c = jax.lax.broadcasted_iota(jnp.int32, (_POOL_ROWS, seq_len), 0)
    s = jax.lax.broadcasted_iota(jnp.int32, (_POOL_ROWS, seq_len), 1)
    in_win = (s >= c * STRIDE) & (s < c * STRIDE + CHUNK) & (c < n_chunks)
    pmat = jnp.where(in_win, 1.0 / CHUNK, 0.0)
    pmat = pmat + jnp.where(c == n_chunks, 1.0 / seq_len, 0.0)
    x_ref[0] = jnp.dot(pmat, hsb, preferred_element_type=jnp.float32)


def _retrieve_body(x_ref, wq_ref, bq_ref, w1_ref, b1_ref, w2_ref, b2_ref,
                   lng_ref, lnb_ref, wf1b_ref, bf1_ref, wgb_ref, bg_ref,
                   f1add_ref, gadd_ref, *, n_chunks, batch):
    X = x_ref[...]
    rows = batch * _POOL_ROWS
    Xf = X.reshape(rows, H)
    hmean = X[:, n_chunks, :]
    qpool = jnp.dot(hmean, wq_ref[...], preferred_element_type=jnp.float32)
    qpool = qpool + bq_ref[...]
    row_i = jax.lax.broadcasted_iota(jnp.int32, (rows, H), 0)
    qexp = jnp.broadcast_to(qpool[:, None, :], (batch, _POOL_ROWS, H))
    Xe = jnp.where((row_i % _POOL_ROWS) == n_chunks, qexp.reshape(rows, H), Xf)
    h = jnp.dot(Xe, w1_ref[...], preferred_element_type=jnp.float32)
    h = _gelu_exact(h + b1_ref[...])
    e = jnp.dot(h, w2_ref[...], preferred_element_type=jnp.float32) + b2_ref[...]
    mu = jnp.mean(e, axis=-1, keepdims=True)
    var = jnp.mean((e - mu) * (e - mu), axis=-1, keepdims=True)
    e = (e - mu) * jax.lax.rsqrt(var + 1e-5) * lng_ref[...] + lnb_ref[...]
    nrm = jnp.maximum(jnp.sqrt(jnp.sum(e * e, axis=-1, keepdims=True)), 1e-12)
    en = e / nrm
    qe = en.reshape(batch, _POOL_ROWS, EMB)[:, n_chunks, :]
    sim = jax.lax.dot_general(qe, en, (((1,), (1,)), ((), ())),
                              preferred_element_type=jnp.float32)
    col = jax.lax.broadcasted_iota(jnp.int32, (batch, rows), 1)
    colf = col.astype(jnp.float32)
    s = jnp.where((col % _POOL_ROWS) < n_chunks, sim, -1e30)
    scores, onehots = [], []
    for _ in range(TOPK):
        m = jnp.max(s, axis=1, keepdims=True)
        first = jnp.min(jnp.where(s >= m, colf, 1e9), axis=1, keepdims=True)
        oh = (colf == first).astype(jnp.float32)
        scores.append(m)
        onehots.append(oh)
        s = jnp.where(oh > 0.0, -1e30, s)
    es = [jnp.exp(sc - scores[0]) for sc in scores]
    z = es[0] + es[1] + es[2]
    ohw = sum((e_ / z) * oh for e_, oh in zip(es, onehots))
    ret = jnp.dot(ohw, Xf, preferred_element_type=jnp.float32)
    f1add = jnp.dot(ret.astype(jnp.bfloat16), wf1b_ref[...],
                    preferred_element_type=jnp.float32)
    f1add_ref[...] = f1add + bf1_ref[...]
    gl = jnp.sum(ret * wgb_ref[...], axis=1, keepdims=True) + bg_ref[0, 0]
    gadd_ref[...] = jnp.broadcast_to(gl, (batch, 128))


def _fusion_body(hs_ref, f1_ref, gadd_ref, wgt_ref, wf1_ref, wf2_ref, bf2_ref,
                 out_ref):
    x = hs_ref[0]
    a = jnp.dot(x.astype(jnp.bfloat16), wf1_ref[...],
                preferred_element_type=jnp.float32)
    a = a + f1_ref[0]
    hgelu = _gelu_exact(a)
    f = jnp.dot(hgelu.astype(jnp.bfloat16), wf2_ref[...],
                preferred_element_type=jnp.float32) + bf2_ref[...]
    gl = jnp.sum(x * wgt_ref[...], axis=1, keepdims=True) + gadd_ref[0][0, 0]
    out_ref[0] = x + jax.nn.sigmoid(gl) * f


def kernel(hidden_states, W1, b1, W2, b2, ln_g, ln_b, Wq, bq, Wg, bg,
           Wf1, bf1, Wf2, bf2):
    B, S, Hd = hidden_states.shape
    n_chunks = (S - CHUNK) // STRIDE + 1
    f32 = jnp.float32

    X = pl.pallas_call(
        functools.partial(_pool_body, n_chunks=n_chunks, seq_len=S),
        grid=(B,),
        in_specs=[pl.BlockSpec((1, S, Hd), lambda b: (b, 0, 0))],
        out_specs=pl.BlockSpec((1, _POOL_ROWS, Hd), lambda b: (b, 0, 0)),
        out_shape=jax.ShapeDtypeStruct((B, _POOL_ROWS, Hd), f32),
    )(hidden_states)

    f1add, gadd = pl.pallas_call(
        functools.partial(_retrieve_body, n_chunks=n_chunks, batch=B),
        out_shape=(jax.ShapeDtypeStruct((B, Hd), f32),
                   jax.ShapeDtypeStruct((B, 128), f32)),
    )(X, Wq, bq.reshape(1, Hd), W1, b1.reshape(1, -1), W2, b2.reshape(1, -1),
      ln_g.reshape(1, -1), ln_b.reshape(1, -1),
      Wf1[Hd:].astype(jnp.bfloat16), bf1.reshape(1, Hd),
      Wg[Hd:, 0].reshape(1, Hd), jnp.broadcast_to(bg.reshape(1, 1), (1, 128)))

    TS = 512
    grid = (B, S // TS)
    out = pl.pallas_call(
        _fusion_body,
        grid=grid,
        in_specs=[
            pl.BlockSpec((1, TS, Hd), lambda b, t: (b, t, 0)),
            pl.BlockSpec((1, 1, Hd), lambda b, t: (b, 0, 0)),
            pl.BlockSpec((1, 1, 128), lambda b, t: (b, 0, 0)),
            pl.BlockSpec((1, Hd), lambda b, t: (0, 0)),
            pl.BlockSpec((Hd, Hd), lambda b, t: (0, 0)),
            pl.BlockSpec((Hd, Hd), lambda b, t: (0, 0)),
            pl.BlockSpec((1, Hd), lambda b, t: (0, 0)),
        ],
        out_specs=pl.BlockSpec((1, TS, Hd), lambda b, t: (b, t, 0)),
        out_shape=jax.ShapeDtypeStruct((B, S, Hd), f32),
        compiler_params=pltpu.CompilerParams(
            dimension_semantics=("parallel", "arbitrary")),
    )(hidden_states, f1add[:, None, :], gadd[:, None, :],
      Wg[:Hd, 0].reshape(1, Hd), Wf1[:Hd].astype(jnp.bfloat16),
      Wf2.astype(jnp.bfloat16), bf2.reshape(1, Hd))
    return out


# speedup vs baseline: 3.6501x; 1.0646x over previous
import functools

import jax
import jax.numpy as jnp
from jax.experimental import pallas as pl
from jax.experimental.pallas import tpu as pltpu

H = 2048
EMB = 128
CHUNK = 64
OVERLAP = 16
TOPK = 3
STRIDE = CHUNK - OVERLAP

_POOL_ROWS = 64

_INV_SQRT2 = 0.7071067811865476


def _gelu_exact(x):
    return 0.5 * x * (1.0 + jax.lax.erf(x * _INV_SQRT2))


def _pool_body(hs_ref, x_ref, *, n_chunks, seq_len, seq_tile):
    t = pl.program_id(1)
    hsb = hs_ref[0]
    c = jax.lax.broadcasted_iota(jnp.int32, (_POOL_ROWS, seq_tile), 0)
    s = jax.lax.broadcasted_iota(jnp.int32, (_POOL_ROWS, seq_tile), 1)
    s = s + t * seq_tile
    in_win = (s >= c * STRIDE) & (s < c * STRIDE + CHUNK) & (c < n_chunks)
    pmat = jnp.where(in_win, 1.0 / CHUNK, 0.0)
    pmat = pmat + jnp.where(c == n_chunks, 1.0 / seq_len, 0.0)
    part = jnp.dot(pmat, hsb, preferred_element_type=jnp.float32)

    @pl.when(t == 0)
    def _init():
        x_ref[0] = part

    @pl.when(t != 0)
    def _acc():
        x_ref[0] += part


def _retrieve_body(x_ref, wq_ref, bq_ref, w1_ref, b1_ref, w2_ref, b2_ref,
                   lng_ref, lnb_ref, wf1b_ref, bf1_ref, wgb_ref, bg_ref,
                   f1add_ref, gadd_ref, *, n_chunks, batch):
    X = x_ref[...]
    rows = batch * _POOL_ROWS
    Xf = X.reshape(rows, H)
    hmean = X[:, n_chunks, :]
    qpool = jnp.dot(hmean, wq_ref[...], preferred_element_type=jnp.float32)
    qpool = qpool + bq_ref[...]
    row_i = jax.lax.broadcasted_iota(jnp.int32, (rows, H), 0)
    qexp = jnp.broadcast_to(qpool[:, None, :], (batch, _POOL_ROWS, H))
    Xe = jnp.where((row_i % _POOL_ROWS) == n_chunks, qexp.reshape(rows, H), Xf)
    h = jnp.dot(Xe, w1_ref[...], preferred_element_type=jnp.float32)
    h = _gelu_exact(h + b1_ref[...])
    e = jnp.dot(h, w2_ref[...], preferred_element_type=jnp.float32) + b2_ref[...]
    mu = jnp.mean(e, axis=-1, keepdims=True)
    var = jnp.mean((e - mu) * (e - mu), axis=-1, keepdims=True)
    e = (e - mu) * jax.lax.rsqrt(var + 1e-5) * lng_ref[...] + lnb_ref[...]
    nrm = jnp.maximum(jnp.sqrt(jnp.sum(e * e, axis=-1, keepdims=True)), 1e-12)
    en = e / nrm
    qe = en.reshape(batch, _POOL_ROWS, EMB)[:, n_chunks, :]
    sim = jax.lax.dot_general(qe, en, (((1,), (1,)), ((), ())),
                              preferred_element_type=jnp.float32)
    col = jax.lax.broadcasted_iota(jnp.int32, (batch, rows), 1)
    colf = col.astype(jnp.float32)
    s = jnp.where((col % _POOL_ROWS) < n_chunks, sim, -1e30)
    scores, onehots = [], []
    for _ in range(TOPK):
        m = jnp.max(s, axis=1, keepdims=True)
        first = jnp.min(jnp.where(s >= m, colf, 1e9), axis=1, keepdims=True)
        oh = (colf == first).astype(jnp.float32)
        scores.append(m)
        onehots.append(oh)
        s = jnp.where(oh > 0.0, -1e30, s)
    es = [jnp.exp(sc - scores[0]) for sc in scores]
    z = es[0] + es[1] + es[2]
    ohw = sum((e_ / z) * oh for e_, oh in zip(es, onehots))
    ret = jnp.dot(ohw, Xf, preferred_element_type=jnp.float32)
    f1add = jnp.dot(ret, wf1b_ref[...], preferred_element_type=jnp.float32)
    f1add_ref[...] = f1add + bf1_ref[...]
    gl = jnp.sum(ret * wgb_ref[...], axis=1, keepdims=True) + bg_ref[0, 0]
    gadd_ref[...] = jnp.broadcast_to(gl, (batch, 128))


def _fusion_body(hs_ref, f1_ref, gadd_ref, wgt_ref, wf1_ref, wf2_ref, bf2_ref,
                 out_ref, wf1bf_ref, wf2bf_ref):
    b, t = pl.program_id(0), pl.program_id(1)

    @pl.when((b == 0) & (t == 0))
    def _cast_weights():
        wf1bf_ref[...] = wf1_ref[...].astype(jnp.bfloat16)
        wf2bf_ref[...] = wf2_ref[...].astype(jnp.bfloat16)

    x = hs_ref[0]
    a = jnp.dot(x.astype(jnp.bfloat16), wf1bf_ref[...],
                preferred_element_type=jnp.float32)
    a = a + f1_ref[0]
    hgelu = _gelu_exact(a)
    f = jnp.dot(hgelu.astype(jnp.bfloat16), wf2bf_ref[...],
                preferred_element_type=jnp.float32) + bf2_ref[...]
    gl = jnp.sum(x * wgt_ref[...], axis=1, keepdims=True) + gadd_ref[0][0, 0]
    out_ref[0] = x + jax.nn.sigmoid(gl) * f


def kernel(hidden_states, W1, b1, W2, b2, ln_g, ln_b, Wq, bq, Wg, bg,
           Wf1, bf1, Wf2, bf2):
    B, S, Hd = hidden_states.shape
    n_chunks = (S - CHUNK) // STRIDE + 1
    f32 = jnp.float32

    PT = 512
    X = pl.pallas_call(
        functools.partial(_pool_body, n_chunks=n_chunks, seq_len=S,
                          seq_tile=PT),
        grid=(B, S // PT),
        in_specs=[pl.BlockSpec((1, PT, Hd), lambda b, t: (b, t, 0))],
        out_specs=pl.BlockSpec((1, _POOL_ROWS, Hd), lambda b, t: (b, 0, 0)),
        out_shape=jax.ShapeDtypeStruct((B, _POOL_ROWS, Hd), f32),
    )(hidden_states)

    f1add, gadd = pl.pallas_call(
        functools.partial(_retrieve_body, n_chunks=n_chunks, batch=B),
        grid=(1,),
        in_specs=[
            pl.BlockSpec((B, _POOL_ROWS, Hd), lambda i: (0, 0, 0)),
            pl.BlockSpec((Hd, Hd), lambda i: (0, 0)),
            pl.BlockSpec((1, Hd), lambda i: (0, 0)),
            pl.BlockSpec((Hd, Hd // 2), lambda i: (0, 0)),
            pl.BlockSpec((1, Hd // 2), lambda i: (0, 0)),
            pl.BlockSpec((Hd // 2, EMB), lambda i: (0, 0)),
            pl.BlockSpec((1, EMB), lambda i: (0, 0)),
            pl.BlockSpec((1, EMB), lambda i: (0, 0)),
            pl.BlockSpec((1, EMB), lambda i: (0, 0)),
            pl.BlockSpec((Hd, Hd), lambda i: (1, 0)),
            pl.BlockSpec((1, Hd), lambda i: (0, 0)),
            pl.BlockSpec((1, Hd), lambda i: (0, 0)),
            pl.BlockSpec((1, 128), lambda i: (0, 0)),
        ],
        out_specs=(pl.BlockSpec((B, Hd), lambda i: (0, 0)),
                   pl.BlockSpec((B, 128), lambda i: (0, 0))),
        out_shape=(jax.ShapeDtypeStruct((B, Hd), f32),
                   jax.ShapeDtypeStruct((B, 128), f32)),
    )(X, Wq, bq.reshape(1, Hd), W1, b1.reshape(1, -1), W2, b2.reshape(1, -1),
      ln_g.reshape(1, -1), ln_b.reshape(1, -1),
      Wf1, bf1.reshape(1, Hd),
      Wg[Hd:, 0].reshape(1, Hd), jnp.broadcast_to(bg.reshape(1, 1), (1, 128)))

    TS = 256
    grid = (B, S // TS)
    out = pl.pallas_call(
        _fusion_body,
        grid=grid,
        in_specs=[
            pl.BlockSpec((1, TS, Hd), lambda b, t: (b, t, 0)),
            pl.BlockSpec((1, 1, Hd), lambda b, t: (b, 0, 0)),
            pl.BlockSpec((1, 1, 128), lambda b, t: (b, 0, 0)),
            pl.BlockSpec((1, Hd), lambda b, t: (0, 0)),
            pl.BlockSpec((Hd, Hd), lambda b, t: (0, 0)),
            pl.BlockSpec((Hd, Hd), lambda b, t: (0, 0)),
            pl.BlockSpec((1, Hd), lambda b, t: (0, 0)),
        ],
        out_specs=pl.BlockSpec((1, TS, Hd), lambda b, t: (b, t, 0)),
        out_shape=jax.ShapeDtypeStruct((B, S, Hd), f32),
        scratch_shapes=[pltpu.VMEM((Hd, Hd), jnp.bfloat16),
                        pltpu.VMEM((Hd, Hd), jnp.bfloat16)],
    )(hidden_states, f1add[:, None, :], gadd[:, None, :],
      Wg[:Hd, 0].reshape(1, Hd), Wf1, Wf2, bf2.reshape(1, Hd))
    return out
